# baseline (speedup 1.0000x reference)
import functools

import jax
import jax.numpy as jnp
from jax import lax
from jax.experimental import pallas as pl
from jax.experimental.pallas import tpu as pltpu
from jax.experimental.pallas import tpu_sc as plsc

B = 4096
N_CAT = 26
CARD = 1000
H = 768
PL_ = 2
N_NUM = 1
K_CAT = N_CAT * PL_
K_ALL = K_CAT + PL_
WIN = 64


def _num_body(x_ref, w_ref, b_ref, o_ref):
    for p in range(PL_):
        o_ref[p] = x_ref[...] * w_ref[p:p + 1] + b_ref[p:p + 1]


def _num_embeds(x_num, W_num, b_num):
    BLK = 512
    return pl.pallas_call(
        _num_body,
        grid=(B // BLK,),
        in_specs=[
            pl.BlockSpec((BLK, N_NUM), lambda i: (i, 0)),
            pl.BlockSpec((PL_, H), lambda i: (0, 0)),
            pl.BlockSpec((PL_, H), lambda i: (0, 0)),
        ],
        out_specs=pl.BlockSpec((PL_, BLK, H), lambda i: (0, i, 0)),
        out_shape=jax.ShapeDtypeStruct((PL_, B, H), jnp.float32),
    )(x_num, W_num.reshape(PL_, H), b_num.reshape(PL_, H))


def _sc_gather(tables_flat, idx, num_planes):
    mesh = plsc.VectorSubcoreMesh(core_axis_name="c", subcore_axis_name="s")

    @functools.partial(
        pl.kernel,
        out_type=jax.ShapeDtypeStruct((K_ALL, B, H), jnp.float32),
        mesh=mesh,
        compiler_params=pltpu.CompilerParams(use_tc_tiling_on_sc=False),
    )
    def kern(tables_hbm, idx_hbm, num_hbm, out_hbm):
        wid = lax.axis_index("s") * 2 + lax.axis_index("c")
        rows = B // 32
        for p in range(PL_):
            pltpu.sync_copy(num_hbm.at[p].at[pl.ds(wid * rows, rows)],
                            out_hbm.at[p].at[pl.ds(wid * rows, rows)])

        def body(idx_vm, o_vm):
            pltpu.sync_copy(tables_hbm.at[idx_vm.at[0]], o_vm.at[0])

        pltpu.emit_pipeline(
            body,
            grid=(K_CAT, B // WIN),
            in_specs=[pl.BlockSpec((1, WIN), lambda k, w: (k, w))],
            out_specs=[pl.BlockSpec((1, WIN, H), lambda k, w: (k + PL_, w, 0))],
            core_axis_name=("c", "s"),
            dimension_semantics=(pltpu.PARALLEL, pltpu.PARALLEL),
        )(idx_hbm, out_hbm)

    return kern(tables_flat, idx, num_planes)


def kernel(x_num, x_cat, W_num, b_num, tables):
    tables_flat = tables.reshape(N_CAT * CARD * PL_, H)
    base = (x_cat.T + (jnp.arange(N_CAT, dtype=jnp.int32) * CARD)[:, None]) * PL_
    idx = jnp.stack([base, base + 1], axis=1).reshape(K_CAT, B)
    num_planes = _num_embeds(x_num, W_num, b_num)
    out = _sc_gather(tables_flat, idx, num_planes)
    return out.transpose(1, 0, 2)


# speedup vs baseline: 3.9224x; 3.9224x over previous
import functools

import jax
import jax.numpy as jnp
from jax import lax
from jax.experimental import pallas as pl
from jax.experimental.pallas import tpu as pltpu
from jax.experimental.pallas import tpu_sc as plsc

B = 4096
N_CAT = 26
CARD = 1000
H = 768
PL_ = 2
N_NUM = 1
D = H * PL_
K_ALL = N_CAT * PL_ + PL_
NW = 32
WINB = 32
NWIN = B // WINB
T_TASKS = N_CAT * NWIN // NW
IPW = T_TASKS * WINB


def _num_body(o_in_ref, x_ref, w_ref, b_ref, o_ref):
    del o_in_ref
    for p in range(PL_):
        o_ref[p] = x_ref[...] * w_ref[p:p + 1] + b_ref[p:p + 1]


def _num_fill(out_sc, x_num, W_num, b_num):
    BLK = 512
    return pl.pallas_call(
        _num_body,
        grid=(B // BLK,),
        in_specs=[
            pl.BlockSpec((PL_, BLK, H), lambda i: (0, i, 0)),
            pl.BlockSpec((BLK, N_NUM), lambda i: (i, 0)),
            pl.BlockSpec((PL_, H), lambda i: (0, 0)),
            pl.BlockSpec((PL_, H), lambda i: (0, 0)),
        ],
        out_specs=pl.BlockSpec((PL_, BLK, H), lambda i: (0, i, 0)),
        out_shape=jax.ShapeDtypeStruct((K_ALL, B, H), jnp.float32),
        input_output_aliases={0: 0},
    )(out_sc, x_num, W_num.reshape(PL_, H), b_num.reshape(PL_, H))


def _sc_gather(tables_flat, idx_flat):
    mesh = plsc.VectorSubcoreMesh(core_axis_name="c", subcore_axis_name="s")

    @functools.partial(
        pl.kernel,
        out_type=jax.ShapeDtypeStruct((K_ALL, B, H), jnp.float32),
        mesh=mesh,
        scratch_types=[
            pltpu.VMEM((IPW,), jnp.int32),
            pltpu.VMEM((WINB, D), jnp.float32),
            pltpu.VMEM((WINB, D), jnp.float32),
            pltpu.SemaphoreType.DMA,
            pltpu.SemaphoreType.DMA,
            pltpu.SemaphoreType.DMA,
            pltpu.SemaphoreType.DMA,
        ],
    )
    def kern(tables_hbm, idx_hbm, out_hbm, idx_v, buf0, buf1,
             gsem0, gsem1, wsem0, wsem1):
        wid = lax.axis_index("s") * 2 + lax.axis_index("c")
        pltpu.sync_copy(idx_hbm.at[pl.ds(wid * IPW, IPW)], idx_v)

        bufs = (buf0, buf1)
        gsems = (gsem0, gsem1)
        wsems = (wsem0, wsem1)
        base = wid * T_TASKS

        def gather_copy(g, slot):
            return pltpu.make_async_copy(
                tables_hbm.at[idx_v.at[pl.ds(g * WINB, WINB)]],
                bufs[slot], gsems[slot])

        def write_copies(g, slot):
            t = base + g
            f = t // NWIN
            b0 = (t % NWIN) * WINB
            return [
                pltpu.make_async_copy(
                    bufs[slot].at[:, pl.ds(p * H, H)],
                    out_hbm.at[PL_ + PL_ * f + p].at[pl.ds(b0, WINB)],
                    wsems[slot])
                for p in range(PL_)
            ]

        gather_copy(0, 0).start()

        @pl.loop(0, T_TASKS // 2)
        def _(h):
            g = h * 2
            gather_copy(g, 0).wait()

            @pl.when(h > 0)
            def _():
                for c in write_copies(g, 1):
                    c.wait()

            for c in write_copies(g, 0):
                c.start()
            gather_copy(g + 1, 1).start()

            gather_copy(g + 1, 1).wait()
            for c in write_copies(g, 0):
                c.wait()
            for c in write_copies(g + 1, 1):
                c.start()

            @pl.when(g + 2 < T_TASKS)
            def _():
                gather_copy(g + 2, 0).start()

        for c in write_copies(T_TASKS - 1, 1):
            c.wait()

    return kern(tables_flat, idx_flat)


def kernel(x_num, x_cat, W_num, b_num, tables):
    tables_flat = tables.reshape(N_CAT * CARD, D)
    idx_flat = (x_cat.T
                + (jnp.arange(N_CAT, dtype=jnp.int32) * CARD)[:, None]
                ).reshape(-1)
    out_sc = _sc_gather(tables_flat, idx_flat)
    out = _num_fill(out_sc, x_num, W_num, b_num)
    return out.transpose(1, 0, 2)
